# baseline (speedup 1.0000x reference)
import functools

import jax
import jax.numpy as jnp
from jax import lax
from jax.experimental import pallas as pl
from jax.experimental.pallas import tpu as pltpu
from jax.experimental.pallas import tpu_sc as plsc

D_MODEL = 64
LANES = 16
NBUF = 2


@jax.jit
def _gather_sc(x, pe_t):
    bsz, t_len = x.shape
    info = plsc.get_sparse_core_info()
    nc, ns = info.num_cores, info.num_subcores
    nw = nc * ns
    bw = bsz // nw
    dt = D_MODEL // 8

    mesh = plsc.VectorSubcoreMesh(core_axis_name="c", subcore_axis_name="s")

    @functools.partial(
        pl.kernel,
        mesh=mesh,
        out_type=jax.ShapeDtypeStruct((t_len, dt, nw, 8, bw), jnp.float32),
        scratch_types=[
            pltpu.VMEM(pe_t.shape, jnp.float32),
            pltpu.VMEM((bw, t_len), jnp.int32),
            pltpu.VMEM((NBUF, dt, 8, bw), jnp.float32),
            pltpu.SemaphoreType.DMA((NBUF,)),
        ],
        compiler_params=pltpu.CompilerParams(
            use_tc_tiling_on_sc=False, needs_layout_passes=False
        ),
    )
    def k(pet_hbm, idx_hbm, out_hbm, pet_v, xb_v, obuf, osem):
        wid = lax.axis_index("s") * nc + lax.axis_index("c")
        pltpu.sync_copy(pet_hbm, pet_v)
        pltpu.sync_copy(idx_hbm.at[pl.ds(wid * bw, bw)], xb_v)

        def out_copy(t, b):
            return pltpu.make_async_copy(
                obuf.at[b], out_hbm.at[t, :, wid], osem.at[b]
            )

        lane = lax.iota(jnp.int32, LANES)

        def fill(t, b):
            t_vec = jnp.full((LANES,), t, jnp.int32)
            for k_ in range(bw // LANES):
                idx = plsc.load_gather(xb_v, [k_ * LANES + lane, t_vec])
                for dr in range(dt):
                    for di in range(8):
                        d_vec = jnp.full((LANES,), dr * 8 + di, jnp.int32)
                        vals = plsc.load_gather(pet_v, [d_vec, idx])
                        obuf[b, dr, di, pl.ds(k_ * LANES, LANES)] = vals

        def body(i, carry):
            for b in range(NBUF):
                t = i * NBUF + b

                @pl.when(t >= NBUF)
                def _():
                    out_copy(t - NBUF, b).wait()

                fill(t, b)
                out_copy(t, b).start()
            return carry

        lax.fori_loop(0, t_len // NBUF, body, 0)

        for b in range(NBUF):
            out_copy(t_len - NBUF + b, b).wait()

    out5 = k(pe_t, x)
    return jnp.transpose(out5, (2, 4, 0, 1, 3)).reshape(bsz, t_len, D_MODEL)


def kernel(x, pe):
    return _gather_sc(x.astype(jnp.int32), pe.T)


# speedup vs baseline: 2.1487x; 2.1487x over previous
import functools

import jax
import jax.numpy as jnp
from jax import lax
from jax.experimental import pallas as pl
from jax.experimental.pallas import tpu as pltpu
from jax.experimental.pallas import tpu_sc as plsc

D_MODEL = 64
LANES = 16
NBUF = 2


@jax.jit
def _gather_sc(x, pe_t):
    bsz, t_len = x.shape
    info = plsc.get_sparse_core_info()
    nc, ns = info.num_cores, info.num_subcores
    nw = nc * ns
    bw = bsz // nw
    dt = D_MODEL // 8

    mesh = plsc.VectorSubcoreMesh(core_axis_name="c", subcore_axis_name="s")

    @functools.partial(
        pl.kernel,
        mesh=mesh,
        out_type=jax.ShapeDtypeStruct((t_len, dt, nw, 8, bw), jnp.float32),
        scratch_types=[
            pltpu.VMEM(pe_t.shape, jnp.float32),
            pltpu.VMEM((bw, t_len), jnp.int32),
            pltpu.VMEM((NBUF, dt, 8, bw), jnp.float32),
            pltpu.SemaphoreType.DMA((NBUF,)),
        ],
        compiler_params=pltpu.CompilerParams(
            use_tc_tiling_on_sc=False, needs_layout_passes=False
        ),
    )
    def k(pet_hbm, idx_hbm, out_hbm, pet_v, xb_v, obuf, osem):
        wid = lax.axis_index("s") * nc + lax.axis_index("c")
        pltpu.sync_copy(pet_hbm, pet_v)
        pltpu.sync_copy(idx_hbm.at[pl.ds(wid * bw, bw)], xb_v)

        def out_copy(t, b):
            return pltpu.make_async_copy(
                obuf.at[b], out_hbm.at[t, :, wid], osem.at[b]
            )

        lane = lax.iota(jnp.int32, LANES)

        def fill(t, b):
            t_vec = jnp.full((LANES,), t, jnp.int32)
            for k_ in range(bw // LANES):
                idx = plsc.load_gather(xb_v, [k_ * LANES + lane, t_vec])
                for dr in range(dt):
                    vals = [
                        plsc.load_gather(
                            pet_v, [jnp.full((LANES,), dr * 8 + di, jnp.int32), idx]
                        )
                        for di in range(8)
                    ]
                    for di in range(8):
                        obuf[b, dr, di, pl.ds(k_ * LANES, LANES)] = vals[di]

        def body(i, carry):
            for b in range(NBUF):
                t = i * NBUF + b

                @pl.when(t >= NBUF)
                def _():
                    out_copy(t - NBUF, b).wait()

                fill(t, b)
                out_copy(t, b).start()
            return carry

        lax.fori_loop(0, t_len // NBUF, body, 0)

        for b in range(NBUF):
            out_copy(t_len - NBUF + b, b).wait()

    out5 = k(pe_t, x)
    return jnp.transpose(out5, (2, 4, 0, 1, 3)).reshape(bsz, t_len, D_MODEL)


def kernel(x, pe):
    return _gather_sc(x.astype(jnp.int32), pe.T)


# speedup vs baseline: 4.3917x; 2.0439x over previous
import functools

import jax
import jax.numpy as jnp
from jax import lax
from jax.experimental import pallas as pl
from jax.experimental.pallas import tpu as pltpu
from jax.experimental.pallas import tpu_sc as plsc

D_MODEL = 64
LANES = 16
NBUF = 2


@jax.jit
def _gather_sc(x, pe_t):
    bsz, t_len = x.shape
    info = plsc.get_sparse_core_info()
    nc, ns = info.num_cores, info.num_subcores
    nw = nc * ns
    bw = bsz // nw
    dt = D_MODEL // 8

    mesh = plsc.VectorSubcoreMesh(core_axis_name="c", subcore_axis_name="s")

    @functools.partial(
        pl.kernel,
        mesh=mesh,
        out_type=jax.ShapeDtypeStruct((t_len, dt, nw, 8, bw), jnp.float32),
        scratch_types=[
            pltpu.VMEM(pe_t.shape, jnp.float32),
            pltpu.VMEM((bw, t_len), jnp.int32),
            pltpu.VMEM((NBUF, dt, 8, bw), jnp.float32),
            pltpu.SemaphoreType.DMA((NBUF,)),
        ],
        compiler_params=pltpu.CompilerParams(
            use_tc_tiling_on_sc=False, needs_layout_passes=False
        ),
    )
    def k(pet_hbm, idx_hbm, out_hbm, pet_v, xb_v, obuf, osem):
        wid = lax.axis_index("s") * nc + lax.axis_index("c")
        pltpu.sync_copy(pet_hbm, pet_v)
        pltpu.sync_copy(idx_hbm.at[pl.ds(wid * bw, bw)], xb_v)

        def out_copy(t, b):
            return pltpu.make_async_copy(
                obuf.at[b], out_hbm.at[t, :, wid], osem.at[b]
            )

        lane = lax.iota(jnp.int32, LANES)

        def fill(t, b):
            t_vec = jnp.full((LANES,), t, jnp.int32)
            idxs = [
                plsc.load_gather(xb_v, [k_ * LANES + lane, t_vec])
                for k_ in range(bw // LANES)
            ]
            order = [
                (k_, dr, di)
                for k_ in range(bw // LANES)
                for dr in range(dt)
                for di in range(8)
            ]
            lag = 8
            vals = {}
            for i, (k_, dr, di) in enumerate(order):
                vals[(k_, dr, di)] = plsc.load_gather(
                    pet_v, [jnp.full((LANES,), dr * 8 + di, jnp.int32), idxs[k_]]
                )
                if i >= lag:
                    pk, pdr, pdi = order[i - lag]
                    obuf[b, pdr, pdi, pl.ds(pk * LANES, LANES)] = vals.pop(
                        (pk, pdr, pdi)
                    )
            for k_, dr, di in order[-lag:]:
                obuf[b, dr, di, pl.ds(k_ * LANES, LANES)] = vals.pop((k_, dr, di))

        def body(i, carry):
            for b in range(NBUF):
                t = i * NBUF + b

                @pl.when(t >= NBUF)
                def _():
                    out_copy(t - NBUF, b).wait()

                fill(t, b)
                out_copy(t, b).start()
            return carry

        lax.fori_loop(0, t_len // NBUF, body, 0)

        for b in range(NBUF):
            out_copy(t_len - NBUF + b, b).wait()

    out5 = k(pe_t, x)
    return jnp.transpose(out5, (2, 4, 0, 1, 3)).reshape(bsz, t_len, D_MODEL)


def kernel(x, pe):
    return _gather_sc(x.astype(jnp.int32), pe.T)
